# initial kernel scaffold (unmeasured)
import jax
import jax.numpy as jnp
from jax import lax
from jax.experimental import pallas as pl
from jax.experimental.pallas import tpu as pltpu


def kernel(
    x,
):
    def body(*refs):
        pass

    out_shape = jax.ShapeDtypeStruct(..., jnp.float32)
    return pl.pallas_call(body, out_shape=out_shape)(...)


# baseline (device time: 1067759 ns/iter reference)
import jax
import jax.numpy as jnp
from jax import lax
from jax.experimental import pallas as pl
from jax.experimental.pallas import tpu as pltpu

M_PER = 8192
N_PER = 1024


def kernel(x):
    m, n = x.shape
    assert (m, n) == (M_PER, 2 * N_PER), (m, n)

    def body(x_ref, out_ref, send_sem, recv_sem, copy_sem):
        my_x = lax.axis_index("x")
        my_y = lax.axis_index("y")
        my_z = lax.axis_index("z")
        peer_z = 1 - my_z

        barrier_sem = pltpu.get_barrier_semaphore()
        pl.semaphore_signal(
            barrier_sem, inc=1,
            device_id=(my_x, my_y, peer_z),
            device_id_type=pl.DeviceIdType.MESH,
        )
        pl.semaphore_wait(barrier_sem, 1)

        rdma = pltpu.make_async_remote_copy(
            src_ref=x_ref.at[:, pl.ds(peer_z * N_PER, N_PER)],
            dst_ref=out_ref.at[pl.ds(my_z * M_PER, M_PER), :],
            send_sem=send_sem,
            recv_sem=recv_sem,
            device_id=(my_x, my_y, peer_z),
            device_id_type=pl.DeviceIdType.MESH,
        )
        rdma.start()

        local = pltpu.make_async_copy(
            x_ref.at[:, pl.ds(my_z * N_PER, N_PER)],
            out_ref.at[pl.ds(my_z * M_PER, M_PER), :],
            copy_sem,
        )
        local.start()

        local.wait()
        rdma.wait()

    return pl.pallas_call(
        body,
        out_shape=jax.ShapeDtypeStruct((2 * M_PER, N_PER), x.dtype),
        in_specs=[pl.BlockSpec(memory_space=pl.ANY)],
        out_specs=pl.BlockSpec(memory_space=pl.ANY),
        scratch_shapes=[
            pltpu.SemaphoreType.DMA,
            pltpu.SemaphoreType.DMA,
            pltpu.SemaphoreType.DMA,
        ],
        compiler_params=pltpu.CompilerParams(collective_id=0),
    )(x)


# device time: 409211 ns/iter; 2.6093x vs baseline; 2.6093x over previous
import jax
import jax.numpy as jnp
from jax import lax
from jax.experimental import pallas as pl
from jax.experimental.pallas import tpu as pltpu

M_PER = 8192
N_PER = 1024
CH = 8
ROWS = M_PER // CH


def kernel(x):
    m, n = x.shape
    assert (m, n) == (M_PER, 2 * N_PER), (m, n)

    def body(x_ref, out_ref, vbuf, lsem, stsem, ssem, rsem):
        my_x = lax.axis_index("x")
        my_y = lax.axis_index("y")
        my_z = lax.axis_index("z")
        peer = (my_x, my_y, 1 - my_z)

        barrier_sem = pltpu.get_barrier_semaphore()
        pl.semaphore_signal(
            barrier_sem, inc=1,
            device_id=peer, device_id_type=pl.DeviceIdType.MESH,
        )
        pl.semaphore_wait(barrier_sem, 1)

        rdma = pltpu.make_async_remote_copy(
            src_ref=x_ref.at[:, pl.ds((1 - my_z) * N_PER, N_PER)],
            dst_ref=out_ref.at[pl.ds(my_z * M_PER, M_PER), :],
            send_sem=ssem,
            recv_sem=rsem,
            device_id=peer,
            device_id_type=pl.DeviceIdType.MESH,
        )
        rdma.start()

        loads = []
        for c in range(CH):
            loads.append(
                pltpu.make_async_copy(
                    x_ref.at[pl.ds(c * ROWS, ROWS), pl.ds(my_z * N_PER, N_PER)],
                    vbuf.at[c % 2],
                    lsem.at[c % 2],
                )
            )
        loads[0].start()
        loads[1].start()
        stores = []
        for c in range(CH):
            loads[c].wait()
            st = pltpu.make_async_copy(
                vbuf.at[c % 2],
                out_ref.at[pl.ds(my_z * M_PER + c * ROWS, ROWS), :],
                stsem.at[c % 2],
            )
            st.start()
            stores.append(st)
            if c + 2 < CH:
                stores[c].wait()
                loads[c + 2].start()
        stores[-2].wait()
        stores[-1].wait()

        rdma.wait()

    return pl.pallas_call(
        body,
        out_shape=jax.ShapeDtypeStruct((2 * M_PER, N_PER), x.dtype),
        in_specs=[pl.BlockSpec(memory_space=pl.ANY)],
        out_specs=pl.BlockSpec(memory_space=pl.ANY),
        scratch_shapes=[
            pltpu.VMEM((2, ROWS, N_PER), jnp.float32),
            pltpu.SemaphoreType.DMA((2,)),
            pltpu.SemaphoreType.DMA((2,)),
            pltpu.SemaphoreType.DMA,
            pltpu.SemaphoreType.DMA,
        ],
        compiler_params=pltpu.CompilerParams(collective_id=0),
    )(x)
